# baseline (speedup 1.0000x reference)
import functools

import jax
import jax.numpy as jnp
import numpy as np
from jax import lax
from jax.experimental import pallas as pl
from jax.experimental.pallas import tpu as pltpu
from jax.experimental.pallas import tpu_sc as plsc

_B, _H, _U, _LK = 4, 16, 64, 4096
_ROWS = _B * _H * _U
_NRES = 8
_BANKW = 2 * _LK

_BANK = (np.arange(_BANKW)[None, :] + np.arange(_NRES)[:, None]) >= _LK

_info = plsc.get_sparse_core_info()
_NC, _NS = _info.num_cores, _info.num_subcores
_NW = _NC * _NS
_RPW = _ROWS // _NW
_CHUNK = 16


@functools.partial(
    pl.kernel,
    mesh=plsc.VectorSubcoreMesh(core_axis_name="c", subcore_axis_name="s"),
    out_type=jax.ShapeDtypeStruct((_ROWS * _LK,), jnp.bool_),
    scratch_types=[
        pltpu.VMEM((_RPW,), jnp.int32),
        pltpu.VMEM((_NRES * _BANKW,), jnp.bool_),
        pltpu.SemaphoreType.DMA,
    ],
)
def _probmask_sc(idx_hbm, bank_hbm, out_hbm, idx_v, bank_v, sem):
    cid = lax.axis_index("c")
    sid = lax.axis_index("s")
    wid = sid * _NC + cid

    pltpu.sync_copy(bank_hbm, bank_v)
    pltpu.sync_copy(idx_hbm.at[pl.ds(wid * _RPW, _RPW)], idx_v)

    def chunk(c, carry):
        idx16 = idx_v[pl.ds(c * _CHUNK, _CHUNK)]
        s16 = (_LK - 1) - idx16
        r16 = lax.rem(s16, _NRES)
        off16 = r16 * _BANKW + (s16 - r16)
        copies = []
        for j in range(_CHUNK):
            row = c * _CHUNK + j
            src_off = pl.multiple_of(off16[j], _NRES)
            dst_off = pl.multiple_of((wid * _RPW + row) * _LK, _LK)
            copies.append(
                pltpu.async_copy(
                    bank_v.at[pl.ds(src_off, _LK)],
                    out_hbm.at[pl.ds(dst_off, _LK)],
                    sem,
                )
            )
        for cp in copies:
            cp.wait()
        return carry

    lax.fori_loop(0, _RPW // _CHUNK, chunk, 0)


def kernel(index, scores):
    del scores
    out = _probmask_sc(index.reshape(_ROWS), jnp.asarray(_BANK).reshape(-1))
    return out.reshape(_B, _H, _U, _LK)


# speedup vs baseline: 2.9781x; 2.9781x over previous
import jax
import jax.numpy as jnp
from jax.experimental import pallas as pl
from jax.experimental.pallas import tpu as pltpu

_B, _H, _U, _LK = 4, 16, 64, 4096
_BH = _B * _H


_R = 8


def _mask_kernel(idx_ref, out_ref):
    idx = idx_ref[...].reshape(_R, _U, 1)
    cols = jax.lax.broadcasted_iota(jnp.int32, (_R, _U, _LK), 2)
    out_ref[...] = cols > idx


def kernel(index, scores):
    del scores
    idx3 = index.reshape(_BH, 1, _U)
    out = pl.pallas_call(
        _mask_kernel,
        grid=(_BH // _R,),
        in_specs=[pl.BlockSpec((_R, 1, _U), lambda i: (i, 0, 0))],
        out_specs=pl.BlockSpec((_R, _U, _LK), lambda i: (i, 0, 0)),
        out_shape=jax.ShapeDtypeStruct((_BH, _U, _LK), jnp.bool_),
    )(idx3)
    return out.reshape(_B, _H, _U, _LK)


# speedup vs baseline: 5.5623x; 1.8677x over previous
import jax
import jax.numpy as jnp
from jax.experimental import pallas as pl

_B, _H, _U, _LK = 4, 16, 64, 4096
_BH = _B * _H
_R = 8


def _mask_kernel(idx_ref, out_ref):
    idx = idx_ref[...].reshape(_R, _U, 1)
    cols = jax.lax.broadcasted_iota(jnp.int32, (_R, _U, _LK), 2)
    out_ref[...] = (cols > idx).astype(jnp.int8)


def kernel(index, scores):
    del scores
    idx3 = index.reshape(_BH, 1, _U)
    out = pl.pallas_call(
        _mask_kernel,
        grid=(_BH // _R,),
        in_specs=[pl.BlockSpec((_R, 1, _U), lambda i: (i, 0, 0))],
        out_specs=pl.BlockSpec((_R, _U, _LK), lambda i: (i, 0, 0)),
        out_shape=jax.ShapeDtypeStruct((_BH, _U, _LK), jnp.int8),
    )(idx3)
    return out.reshape(_B, _H, _U, _LK).astype(jnp.bool_)
